# baseline (speedup 1.0000x reference)
import dataclasses
import functools

import jax
import jax.numpy as jnp
from jax.experimental import pallas as pl
from jax.experimental.pallas import tpu as pltpu
from jax.experimental.pallas import tpu_sc as plsc

_T = 1.0
_EPS = 1e-8
_LANES = 16
_CHUNK = 16384
_SC_SHARE = 5
_TOTAL_SHARE = 8
_TC_BLK = 65536


def _bucket(tv, km1):
    x = tv * float(km1)
    xi = x.astype(jnp.int32)
    xf = xi.astype(jnp.float32)
    return jnp.maximum(jnp.where(x == xf, xi - 1, xi), 0)


def _make_sc(n, n_sc, ch, km1):
    mesh = plsc.VectorSubcoreMesh(core_axis_name="c", subcore_axis_name="s")
    h = _T / float(km1)
    inv_denom = 1.0 / (h + _EPS)
    nchunks = km1 // _LANES

    cp = pltpu.CompilerParams()
    if "needs_layout_passes" in pltpu.CompilerParams.__dataclass_fields__:
        cp = dataclasses.replace(cp, needs_layout_passes=False)

    @functools.partial(
        pl.kernel, mesh=mesh,
        out_type=(jax.ShapeDtypeStruct((n,), jnp.float32),
                  jax.ShapeDtypeStruct((n,), jnp.float32)),
        scratch_types=[pltpu.VMEM((km1,), jnp.float32),
                       pltpu.VMEM((km1,), jnp.float32),
                       pltpu.VMEM((km1,), jnp.float32)],
        compiler_params=cp,
    )
    def k(t_hbm, theta_hbm, tau_hbm, dtau_hbm, theta_v, b_v, slope_v):
        pltpu.sync_copy(theta_hbm, theta_v)

        chunks = [theta_v[pl.ds(c * _LANES, _LANES)] for c in range(nchunks)]
        m = jax.lax.reduce_max(chunks[0], (0,))
        for c in range(1, nchunks):
            m = jnp.maximum(m, jax.lax.reduce_max(chunks[c], (0,)))
        es = [jnp.exp(chunks[c] - m) for c in range(nchunks)]
        total = jax.lax.reduce_sum(es[0], (0,))
        for c in range(1, nchunks):
            total = total + jax.lax.reduce_sum(es[c], (0,))
        inv_total = jnp.full((_LANES,), _T, jnp.float32) / total
        lane_f = jax.lax.iota(jnp.int32, _LANES).astype(jnp.float32)
        carry = jnp.float32(0.0)
        for c in range(nchunks):
            cs = plsc.cumsum(es[c]) + carry
            y0 = (cs - es[c]) * inv_total
            s = es[c] * inv_total * inv_denom
            t0 = (lane_f + float(c * _LANES)) * h
            b_v[pl.ds(c * _LANES, _LANES)] = y0 - t0 * s
            slope_v[pl.ds(c * _LANES, _LANES)] = s
            carry = carry + jax.lax.reduce_sum(es[c], (0,))

        def body(t_vmem, tau_vmem, dtau_vmem):
            @plsc.parallel_loop(0, ch, step=_LANES, unroll=8)
            def _(i):
                tv = t_vmem[pl.ds(i, _LANES)]
                j = _bucket(tv, km1)
                b = plsc.load_gather(b_v, [j])
                s = plsc.load_gather(slope_v, [j])
                tau_vmem[pl.ds(i, _LANES)] = b + tv * s
                dtau_vmem[pl.ds(i, _LANES)] = s

        pltpu.emit_pipeline(
            body,
            grid=(n_sc // ch,),
            in_specs=[pl.BlockSpec((ch,), lambda i: (i,))],
            out_specs=[pl.BlockSpec((ch,), lambda i: (i,)),
                       pl.BlockSpec((ch,), lambda i: (i,))],
            core_axis_name=("c", "s"),
            dimension_semantics=(pltpu.PARALLEL,),
        )(t_hbm, tau_hbm, dtau_hbm)

    return k


def _tc_table(theta, km1):
    m = jnp.max(theta)
    e = jnp.exp(theta - m)
    w = e / jnp.sum(e)
    inc = w * _T
    row = jax.lax.broadcasted_iota(jnp.int32, (km1, km1), 0)
    col = jax.lax.broadcasted_iota(jnp.int32, (km1, km1), 1)
    tri = jnp.where(row <= col, 1.0, 0.0).astype(jnp.float32)
    cs = jax.lax.dot_general(inc, tri, (((1,), (0,)), ((), ())),
                             precision=jax.lax.Precision.HIGHEST,
                             preferred_element_type=jnp.float32)
    y0 = cs - inc
    h = _T / float(km1)
    lane = jax.lax.broadcasted_iota(jnp.int32, (1, km1), 1).astype(jnp.float32)
    slope = (cs - y0) * (1.0 / (h + _EPS))
    b = y0 - (lane * h) * slope
    return b, slope


def _make_tc(n, n_sc, km1):
    blk = _TC_BLK
    rows = blk // 128
    off = n_sc // blk
    ntc = (n - n_sc) // blk

    def body(theta_ref, t_ref, tau_a_ref, dtau_a_ref, tau_ref, dtau_ref,
             b_tab, s_tab):
        del tau_a_ref, dtau_a_ref

        @pl.when(pl.program_id(0) == 0)
        def _():
            b, s = _tc_table(theta_ref[...], km1)
            b_tab[...] = b
            s_tab[...] = s

        tv = t_ref[...].reshape(rows, 128)
        j = _bucket(tv, km1)
        bt = jnp.broadcast_to(b_tab[...], (rows, km1))
        st = jnp.broadcast_to(s_tab[...], (rows, km1))
        b = jnp.take_along_axis(bt, j, axis=1, mode="promise_in_bounds")
        s = jnp.take_along_axis(st, j, axis=1, mode="promise_in_bounds")
        tau_ref[...] = (b + tv * s).reshape(blk)
        dtau_ref[...] = s.reshape(blk)

    any_spec = pl.BlockSpec(memory_space=pltpu.MemorySpace.HBM)
    return pl.pallas_call(
        body,
        grid=(ntc,),
        in_specs=[pl.BlockSpec((1, km1), lambda i: (0, 0)),
                  pl.BlockSpec((blk,), lambda i: (i + off,)),
                  any_spec, any_spec],
        out_specs=[pl.BlockSpec((blk,), lambda i: (i + off,)),
                   pl.BlockSpec((blk,), lambda i: (i + off,))],
        out_shape=(jax.ShapeDtypeStruct((n,), jnp.float32),
                   jax.ShapeDtypeStruct((n,), jnp.float32)),
        scratch_shapes=[pltpu.VMEM((1, km1), jnp.float32),
                        pltpu.VMEM((1, km1), jnp.float32)],
        input_output_aliases={2: 0, 3: 1},
    )


def kernel(t, theta, times):
    del times
    n = t.shape[0]
    km1 = theta.shape[0]
    unit = n // _TOTAL_SHARE
    n_sc = unit * _SC_SHARE
    tau1, dtau1 = _make_sc(n, n_sc, _CHUNK, km1)(t, theta)
    tau, dtau = _make_tc(n, n_sc, km1)(theta.reshape(1, km1), t, tau1, dtau1)
    return tau, dtau


# speedup vs baseline: 1.3089x; 1.3089x over previous
import dataclasses
import functools

import jax
import jax.numpy as jnp
from jax.experimental import pallas as pl
from jax.experimental.pallas import tpu as pltpu
from jax.experimental.pallas import tpu_sc as plsc

_T = 1.0
_EPS = 1e-8
_LANES = 16
_CHUNK = 16384


def _make_interp(n, ch, km1):
    mesh = plsc.VectorSubcoreMesh(core_axis_name="c", subcore_axis_name="s")
    scale = float(km1) / _T
    h = _T / float(km1)
    inv_denom = 1.0 / (h + _EPS)
    nchunks = km1 // _LANES

    cp = pltpu.CompilerParams()
    if "needs_layout_passes" in pltpu.CompilerParams.__dataclass_fields__:
        cp = dataclasses.replace(cp, needs_layout_passes=False)

    @functools.partial(
        pl.kernel, mesh=mesh,
        out_type=(jax.ShapeDtypeStruct((n,), jnp.float32),
                  jax.ShapeDtypeStruct((n,), jnp.float32)),
        scratch_types=[pltpu.VMEM((km1,), jnp.float32),
                       pltpu.VMEM((km1,), jnp.float32),
                       pltpu.VMEM((km1,), jnp.float32)],
        compiler_params=cp,
    )
    def k(t_hbm, theta_hbm, tau_hbm, dtau_hbm, theta_v, b_v, slope_v):
        pltpu.sync_copy(theta_hbm, theta_v)

        chunks = [theta_v[pl.ds(c * _LANES, _LANES)] for c in range(nchunks)]
        m = jax.lax.reduce_max(chunks[0], (0,))
        for c in range(1, nchunks):
            m = jnp.maximum(m, jax.lax.reduce_max(chunks[c], (0,)))
        es = [jnp.exp(chunks[c] - m) for c in range(nchunks)]
        total = jax.lax.reduce_sum(es[0], (0,))
        for c in range(1, nchunks):
            total = total + jax.lax.reduce_sum(es[c], (0,))
        inv_total = jnp.full((_LANES,), _T, jnp.float32) / total
        lane_f = jax.lax.iota(jnp.int32, _LANES).astype(jnp.float32)
        carry = jnp.float32(0.0)
        for c in range(nchunks):
            cs = plsc.cumsum(es[c]) + carry
            y0 = (cs - es[c]) * inv_total
            s = es[c] * inv_total * inv_denom
            t0 = (lane_f + float(c * _LANES)) * h
            b_v[pl.ds(c * _LANES, _LANES)] = y0 - t0 * s
            slope_v[pl.ds(c * _LANES, _LANES)] = s
            carry = carry + jax.lax.reduce_sum(es[c], (0,))

        def body(t_vmem, tau_vmem, dtau_vmem):
            @plsc.parallel_loop(0, ch, step=_LANES, unroll=8)
            def _(i):
                tv = t_vmem[pl.ds(i, _LANES)]
                x = tv * scale
                xi = x.astype(jnp.int32)
                xf = xi.astype(jnp.float32)
                j = jnp.maximum(jnp.where(x == xf, xi - 1, xi), 0)
                b = plsc.load_gather(b_v, [j])
                s = plsc.load_gather(slope_v, [j])
                tau_vmem[pl.ds(i, _LANES)] = b + tv * s
                dtau_vmem[pl.ds(i, _LANES)] = s

        pltpu.emit_pipeline(
            body,
            grid=(n // ch,),
            in_specs=[pl.BlockSpec((ch,), lambda i: (i,))],
            out_specs=[pl.BlockSpec((ch,), lambda i: (i,)),
                       pl.BlockSpec((ch,), lambda i: (i,))],
            core_axis_name=("c", "s"),
            dimension_semantics=(pltpu.PARALLEL,),
        )(t_hbm, tau_hbm, dtau_hbm)

    return k


def kernel(t, theta, times):
    del times
    n = t.shape[0]
    km1 = theta.shape[0]
    tau, dtau = _make_interp(n, _CHUNK, km1)(t, theta)
    return tau, dtau
